# baseline (speedup 1.0000x reference)
import functools

import numpy as np
import jax
import jax.numpy as jnp
from jax import lax
from jax.experimental import pallas as pl
from jax.experimental.pallas import tpu as pltpu
from jax.experimental.pallas import tpu_sc as plsc

_CUT_WIDTH = 64
_STRIPES_NUM = 2

_NC = 2
_NS = 16
_NW = _NC * _NS

_CH = 128
_NBUF = 4
_WIN = 128


@functools.lru_cache(maxsize=None)
def _stripe_tables(batch: int, width: int):
    rng = np.random.default_rng(0)
    perm = rng.permutation(batch)
    stripes = []
    for _ in range(batch):
        s = []
        for _ in range(_STRIPES_NUM):
            distance = int(rng.integers(0, _CUT_WIDTH))
            bgn = int(rng.integers(0, width - distance))
            s.append((bgn, distance))
        stripes.append(s)

    params = np.zeros((batch, 8), dtype=np.int32)
    n_win = batch * _STRIPES_NUM
    win_rows = np.empty((n_win, _WIN), dtype=np.int32)
    for n in range(batch):
        for s, (bgn, distance) in enumerate(stripes[n]):
            start = min((bgn // 8) * 8, width - _WIN)
            params[n, 4 * s + 0] = start
            params[n, 4 * s + 1] = bgn - start
            params[n, 4 * s + 2] = bgn - start + distance
            win_rows[n * _STRIPES_NUM + s] = (
                perm[n] * width + start + np.arange(_WIN, dtype=np.int32))
    win_idx = win_rows.reshape(_NW, -1, _CH)
    return win_idx, params


def _sc_gather(x, idx, rows, feat, n_chunk):
    mesh = plsc.VectorSubcoreMesh(core_axis_name="c", subcore_axis_name="s")

    @functools.partial(
        pl.kernel,
        out_type=jax.ShapeDtypeStruct((rows, feat), jnp.float32),
        mesh=mesh,
        scratch_types=[
            pltpu.VMEM((n_chunk, _CH), jnp.int32),
        ] + [pltpu.VMEM((_CH, feat), jnp.float32)] * _NBUF
          + [pltpu.SemaphoreType.DMA] * (2 * _NBUF),
    )
    def body(x_hbm, idx_hbm, out_hbm, idx_v, *bufs_sems):
        bufs = bufs_sems[:_NBUF]
        gsems = bufs_sems[_NBUF:2 * _NBUF]
        wsems = bufs_sems[2 * _NBUF:]
        wid = lax.axis_index("s") * _NC + lax.axis_index("c")
        base = wid * (n_chunk * _CH)
        pltpu.sync_copy(idx_hbm.at[wid], idx_v)

        def gather(c):
            b = c % _NBUF
            return pltpu.async_copy(x_hbm.at[idx_v.at[c]], bufs[b], gsems[b])

        gh = {c: gather(c) for c in range(min(_NBUF, n_chunk))}
        wh = {}
        for c in range(n_chunk):
            if c > 0 and c - 1 + _NBUF < n_chunk:
                wh[c - 1].wait()
                gh[c - 1 + _NBUF] = gather(c - 1 + _NBUF)
            gh[c].wait()
            wh[c] = pltpu.async_copy(
                bufs[c % _NBUF], out_hbm.at[pl.ds(base + c * _CH, _CH)],
                wsems[c % _NBUF])
        for c in range(max(0, n_chunk - _NBUF), n_chunk):
            wh[c].wait()

    return body(x, idx)


def _tc_merge(x, win, params, batch, width, feat):
    def body(params_ref, x_ref, win_ref, out_ref):
        out_ref[...] = x_ref[...]
        col = lax.broadcasted_iota(jnp.int32, (_WIN, feat), 0)
        for s in range(_STRIPES_NUM):
            start = pl.multiple_of(params_ref[0, 0, 4 * s + 0], 8)
            lo = params_ref[0, 0, 4 * s + 1]
            hi = params_ref[0, 0, 4 * s + 2]
            cur = out_ref[0, pl.ds(start, _WIN), :]
            m = (col >= lo) & (col < hi)
            out_ref[0, pl.ds(start, _WIN), :] = jnp.where(
                m, win_ref[0, s], cur)

    return pl.pallas_call(
        body,
        grid=(batch,),
        in_specs=[
            pl.BlockSpec((1, 1, 8), lambda n: (n, 0, 0),
                         memory_space=pltpu.SMEM),
            pl.BlockSpec((1, width, feat), lambda n: (n, 0, 0)),
            pl.BlockSpec((1, _STRIPES_NUM, _WIN, feat),
                         lambda n: (n, 0, 0, 0)),
        ],
        out_specs=pl.BlockSpec((1, width, feat), lambda n: (n, 0, 0)),
        out_shape=jax.ShapeDtypeStruct((batch, width, feat), jnp.float32),
    )(params.reshape(batch, 1, 8), x, win)


def kernel(input):
    batch, chan, width, feat = input.shape
    rows = batch * chan * width
    win_idx, params = _stripe_tables(batch, width)
    n_win_rows = batch * _STRIPES_NUM * _WIN
    xf = input.reshape(rows, feat)
    win = _sc_gather(xf, jnp.asarray(win_idx), n_win_rows, feat,
                     n_win_rows // (_NW * _CH))
    out = _tc_merge(input.reshape(batch, width, feat),
                    win.reshape(batch, _STRIPES_NUM, _WIN, feat),
                    jnp.asarray(params), batch, width, feat)
    return out.reshape(input.shape)


# speedup vs baseline: 1.3958x; 1.3958x over previous
import functools

import numpy as np
import jax
import jax.numpy as jnp
from jax import lax
from jax.experimental import pallas as pl
from jax.experimental.pallas import tpu as pltpu
from jax.experimental.pallas import tpu_sc as plsc

_CUT_WIDTH = 64
_STRIPES_NUM = 2

_NC = 2
_NS = 16
_NW = _NC * _NS

_CH = 128
_NBUF = 6


@functools.lru_cache(maxsize=None)
def _gather_rows(batch: int, width: int) -> np.ndarray:
    rng = np.random.default_rng(0)
    perm = rng.permutation(batch)
    src = np.tile(np.arange(batch, dtype=np.int64)[:, None], (1, width))
    for n in range(batch):
        for _ in range(_STRIPES_NUM):
            distance = int(rng.integers(0, _CUT_WIDTH))
            bgn = int(rng.integers(0, width - distance))
            if distance:
                src[n, bgn:bgn + distance] = perm[n]
    rows = src * width + np.arange(width)[None, :]
    return rows.reshape(-1).astype(np.int32)


def _sc_gather(x, idx, rows, feat, n_chunk):
    mesh = plsc.VectorSubcoreMesh(core_axis_name="c", subcore_axis_name="s")

    @functools.partial(
        pl.kernel,
        out_type=jax.ShapeDtypeStruct((rows, feat), jnp.float32),
        mesh=mesh,
        scratch_types=[
            pltpu.VMEM((n_chunk, _CH), jnp.int32),
        ] + [pltpu.VMEM((_CH, feat), jnp.float32)] * _NBUF
          + [pltpu.SemaphoreType.DMA] * (2 * _NBUF),
    )
    def body(x_hbm, idx_hbm, out_hbm, idx_v, *bufs_sems):
        bufs = bufs_sems[:_NBUF]
        gsems = bufs_sems[_NBUF:2 * _NBUF]
        wsems = bufs_sems[2 * _NBUF:]
        wid = lax.axis_index("s") * _NC + lax.axis_index("c")
        base = wid * (n_chunk * _CH)
        pltpu.sync_copy(idx_hbm.at[wid], idx_v)

        def gather(c):
            b = c % _NBUF
            return pltpu.async_copy(x_hbm.at[idx_v.at[c]], bufs[b], gsems[b])

        gh = {c: gather(c) for c in range(min(_NBUF, n_chunk))}
        wh = {}
        for c in range(n_chunk):
            if c > 0 and c - 1 + _NBUF < n_chunk:
                wh[c - 1].wait()
                gh[c - 1 + _NBUF] = gather(c - 1 + _NBUF)
            gh[c].wait()
            wh[c] = pltpu.async_copy(
                bufs[c % _NBUF], out_hbm.at[pl.ds(base + c * _CH, _CH)],
                wsems[c % _NBUF])
        for c in range(max(0, n_chunk - _NBUF), n_chunk):
            wh[c].wait()

    return body(x, idx)


def kernel(input):
    batch, chan, width, feat = input.shape
    rows = batch * chan * width
    per_w = rows // _NW
    n_chunk = per_w // _CH
    g = _gather_rows(batch, width).reshape(_NW, n_chunk, _CH)
    x = input.reshape(rows, feat)
    out = _sc_gather(x, jnp.asarray(g), rows, feat, n_chunk)
    return out.reshape(input.shape)


# speedup vs baseline: 1.3980x; 1.0016x over previous
import functools

import numpy as np
import jax
import jax.numpy as jnp
from jax import lax
from jax.experimental import pallas as pl
from jax.experimental.pallas import tpu as pltpu
from jax.experimental.pallas import tpu_sc as plsc

_CUT_WIDTH = 64
_STRIPES_NUM = 2

_NC = 2
_NS = 16
_NW = _NC * _NS

_CH = 128
_NBUF = 7


@functools.lru_cache(maxsize=None)
def _gather_rows(batch: int, width: int) -> np.ndarray:
    rng = np.random.default_rng(0)
    perm = rng.permutation(batch)
    src = np.tile(np.arange(batch, dtype=np.int64)[:, None], (1, width))
    for n in range(batch):
        for _ in range(_STRIPES_NUM):
            distance = int(rng.integers(0, _CUT_WIDTH))
            bgn = int(rng.integers(0, width - distance))
            if distance:
                src[n, bgn:bgn + distance] = perm[n]
    rows = src * width + np.arange(width)[None, :]
    return rows.reshape(-1).astype(np.int32)


def _sc_gather(x, idx, rows, feat, n_chunk):
    mesh = plsc.VectorSubcoreMesh(core_axis_name="c", subcore_axis_name="s")

    @functools.partial(
        pl.kernel,
        out_type=jax.ShapeDtypeStruct((rows, feat), jnp.float32),
        mesh=mesh,
        scratch_types=[
            pltpu.VMEM((n_chunk, _CH), jnp.int32),
        ] + [pltpu.VMEM((_CH, feat), jnp.float32)] * _NBUF
          + [pltpu.SemaphoreType.DMA] * (2 * _NBUF),
    )
    def body(x_hbm, idx_hbm, out_hbm, idx_v, *bufs_sems):
        bufs = bufs_sems[:_NBUF]
        gsems = bufs_sems[_NBUF:2 * _NBUF]
        wsems = bufs_sems[2 * _NBUF:]
        wid = lax.axis_index("s") * _NC + lax.axis_index("c")
        base = wid * (n_chunk * _CH)
        pltpu.sync_copy(idx_hbm.at[wid], idx_v)

        def gather(c):
            b = c % _NBUF
            return pltpu.async_copy(x_hbm.at[idx_v.at[c]], bufs[b], gsems[b])

        gh = {c: gather(c) for c in range(min(_NBUF, n_chunk))}
        wh = {}
        for c in range(n_chunk):
            if c > 0 and c - 1 + _NBUF < n_chunk:
                wh[c - 1].wait()
                gh[c - 1 + _NBUF] = gather(c - 1 + _NBUF)
            gh[c].wait()
            wh[c] = pltpu.async_copy(
                bufs[c % _NBUF], out_hbm.at[pl.ds(base + c * _CH, _CH)],
                wsems[c % _NBUF])
        for c in range(max(0, n_chunk - _NBUF), n_chunk):
            wh[c].wait()

    return body(x, idx)


def kernel(input):
    batch, chan, width, feat = input.shape
    rows = batch * chan * width
    per_w = rows // _NW
    n_chunk = per_w // _CH
    g = _gather_rows(batch, width).reshape(_NW, n_chunk, _CH)
    x = input.reshape(rows, feat)
    out = _sc_gather(x, jnp.asarray(g), rows, feat, n_chunk)
    return out.reshape(input.shape)
